# baseline (speedup 1.0000x reference)
import dataclasses
import functools

import jax
import jax.numpy as jnp
import numpy as np
from jax import lax
from jax.experimental import pallas as pl
from jax.experimental.pallas import tpu as pltpu
from jax.experimental.pallas import tpu_sc as plsc

EPS = float(np.finfo(np.float32).eps)

NTILES = 32
LANES = 16
EBLK = 2560


def _round_up(x, m):
    return (x + m - 1) // m * m


def _sc_compiler_params():
    cp = pltpu.CompilerParams()
    if "needs_layout_passes" in pltpu.CompilerParams.__dataclass_fields__:
        cp = dataclasses.replace(cp, needs_layout_passes=False)
    return cp


NT = (((1,), (1,)), ((), ()))
NN = (((1,), (0,)), ((), ()))


def _hdot(a, b, dims):
    return lax.dot_general(a, b, dims, precision=lax.Precision.HIGHEST,
                           preferred_element_type=jnp.float32)


def _pack_bf16_pair(y):
    half = y.shape[1] // 2
    u = lax.bitcast_convert_type(y, jnp.uint32) + jnp.uint32(0x8000)
    lo = u[:, :half] >> jnp.uint32(16)
    hi = u[:, half:] & jnp.uint32(0xFFFF0000)
    return lax.bitcast_convert_type(lo | hi, jnp.int32)


def _unpack_bf16_pair(v):
    f_lo = lax.bitcast_convert_type(v << jnp.int32(16), jnp.float32)
    f_hi = lax.bitcast_convert_type(
        v & jnp.int32(np.int32(np.uint32(0xFFFF0000))), jnp.float32)
    return f_lo, f_hi


def _tc_prep_body(xs_ref, xt_ref, ws_ref, wt_ref, wa_ref,
                  ys_ref, yt_ref, es_ref, et_ref):
    ys = _hdot(xs_ref[...], ws_ref[...], NT)
    yt = _hdot(xt_ref[...], wt_ref[...], NT)
    ys_ref[...] = _pack_bf16_pair(ys)
    yt_ref[...] = _pack_bf16_pair(yt)
    es_ref[...] = _hdot(wa_ref[...], ys, NT)
    et_ref[...] = _hdot(wa_ref[...], yt, NT)


def _tc_prep(x_s, x_t, W_src, W_tgt, W_attn):
    N, D = x_s.shape
    full = lambda shape: pl.BlockSpec(shape, lambda: (0, 0))
    return pl.pallas_call(
        _tc_prep_body,
        in_specs=[full((N, D)), full((N, D)),
                  full((D, D)), full((D, D)), full((1, D))],
        out_specs=[full((N, D // 2)), full((N, D // 2)),
                   full((1, N)), full((1, N))],
        out_shape=[
            jax.ShapeDtypeStruct((N, D // 2), jnp.int32),
            jax.ShapeDtypeStruct((N, D // 2), jnp.int32),
            jax.ShapeDtypeStruct((1, N), jnp.float32),
            jax.ShapeDtypeStruct((1, N), jnp.float32),
        ],
    )(x_s, x_t, W_src, W_tgt, W_attn)


def _tc_ee_body(ea_ref, we_ref, wa_ref, ee_ref):
    c = _hdot(wa_ref[...], we_ref[...], NN)
    ee_ref[...] = _hdot(c, ea_ref[...], NT)


def _tc_ee(edge_attr, W_edge, W_attn):
    E, D = edge_attr.shape
    return pl.pallas_call(
        _tc_ee_body,
        grid=(E // EBLK,),
        in_specs=[
            pl.BlockSpec((EBLK, D), lambda i: (i, 0)),
            pl.BlockSpec((D, D), lambda i: (0, 0)),
            pl.BlockSpec((1, D), lambda i: (0, 0)),
        ],
        out_specs=pl.BlockSpec((1, EBLK), lambda i: (0, i)),
        out_shape=jax.ShapeDtypeStruct((1, E), jnp.float32),
    )(edge_attr, W_edge, W_attn)


def _sc_logits(es_t, et_t, ee2d, src2d, tgt2d, E):
    npad = es_t.shape[0]
    nslice = npad // LANES
    rows_pad, width = ee2d.shape
    R = rows_pad // NTILES

    mesh = plsc.VectorSubcoreMesh(core_axis_name="core",
                                  subcore_axis_name="subcore")

    @functools.partial(
        pl.kernel, mesh=mesh, compiler_params=_sc_compiler_params(),
        out_type=(jax.ShapeDtypeStruct((rows_pad, width), jnp.float32),
                  jax.ShapeDtypeStruct((2, npad), jnp.float32)),
        scratch_types=[
            pltpu.VMEM((npad,), jnp.float32),
            pltpu.VMEM((npad,), jnp.float32),
            pltpu.VMEM((R, width), jnp.float32),
            pltpu.VMEM((R, width), jnp.int32),
            pltpu.VMEM((R, width), jnp.int32),
            pltpu.VMEM((R, width), jnp.float32),
            pltpu.VMEM((nslice,), jnp.float32),
            pltpu.VMEM_SHARED((npad,), jnp.float32),
        ],
    )
    def k(es_hbm, et_hbm, ee_hbm, src_hbm, tgt_hbm, p_hbm, den_hbm,
          es_v, et_v, ee_v, src_v, tgt_v, p_v, zero_v, den_sh):
        cid = lax.axis_index("core")
        sid = lax.axis_index("subcore")
        w = sid * 2 + cid
        base = w * R

        pltpu.sync_copy(es_hbm, es_v)
        pltpu.sync_copy(et_hbm, et_v)
        pltpu.sync_copy(ee_hbm.at[pl.ds(base, R)], ee_v)
        pltpu.sync_copy(src_hbm.at[pl.ds(base, R)], src_v)
        pltpu.sync_copy(tgt_hbm.at[pl.ds(base, R)], tgt_v)

        @pl.loop(0, nslice, step=LANES)
        def _(i):
            zero_v[pl.ds(i, LANES)] = jnp.zeros((LANES,), jnp.float32)

        pltpu.sync_copy(zero_v, den_sh.at[pl.ds(sid * nslice, nslice)])
        plsc.subcore_barrier()

        @pl.loop(0, R)
        def _(r):
            @pl.loop(0, width, step=LANES)
            def _(col):
                si = src_v[r, pl.ds(col, LANES)]
                ti = tgt_v[r, pl.ds(col, LANES)]
                ev = ee_v[r, pl.ds(col, LANES)]
                e = plsc.load_gather(es_v, [si]) + plsc.load_gather(et_v, [ti]) + ev
                e = jnp.maximum(e, e * jnp.float32(0.2))
                pos = (base + r) * width + col + lax.iota(jnp.int32, LANES)
                p = jnp.where(pos < E, jnp.exp(e), jnp.float32(0.0))
                p_v[r, pl.ds(col, LANES)] = p

        @pl.loop(0, R)
        def _(r):
            pltpu.sync_copy(p_v.at[r], den_sh.at[src_v.at[r]], add=True)

        plsc.subcore_barrier()
        pltpu.sync_copy(p_v, p_hbm.at[pl.ds(base, R)])

        @pl.when(sid == 0)
        def _():
            pltpu.sync_copy(den_sh, den_hbm.at[cid])

    return k(es_t, et_t, ee2d, src2d, tgt2d)


def _sc_alpha(dens, p2d, src2d):
    npad = dens.shape[1]
    rows_pad, width = p2d.shape
    R = rows_pad // NTILES

    mesh = plsc.VectorSubcoreMesh(core_axis_name="core",
                                  subcore_axis_name="subcore")

    @functools.partial(
        pl.kernel, mesh=mesh, compiler_params=_sc_compiler_params(),
        out_type=jax.ShapeDtypeStruct((rows_pad, width), jnp.float32),
        scratch_types=[
            pltpu.VMEM((npad,), jnp.float32),
            pltpu.VMEM((npad,), jnp.float32),
            pltpu.VMEM((R, width), jnp.float32),
            pltpu.VMEM((R, width), jnp.int32),
            pltpu.VMEM((R, width), jnp.float32),
        ],
    )
    def k(den_hbm, p_hbm, src_hbm, a_hbm, d0_v, d1_v, p_v, src_v, a_v):
        cid = lax.axis_index("core")
        sid = lax.axis_index("subcore")
        w = sid * 2 + cid
        base = w * R

        pltpu.sync_copy(den_hbm.at[0], d0_v)
        pltpu.sync_copy(den_hbm.at[1], d1_v)
        pltpu.sync_copy(p_hbm.at[pl.ds(base, R)], p_v)
        pltpu.sync_copy(src_hbm.at[pl.ds(base, R)], src_v)

        @pl.loop(0, npad, step=LANES)
        def _(i):
            d0_v[pl.ds(i, LANES)] = d0_v[pl.ds(i, LANES)] + d1_v[pl.ds(i, LANES)]

        @pl.loop(0, R)
        def _(r):
            @pl.loop(0, width, step=LANES)
            def _(col):
                si = src_v[r, pl.ds(col, LANES)]
                d = plsc.load_gather(d0_v, [si])
                a_v[r, pl.ds(col, LANES)] = p_v[r, pl.ds(col, LANES)] / d

        pltpu.sync_copy(a_v, a_hbm.at[pl.ds(base, R)])

    return k(dens, p2d, src2d)


def _sc_gather(ys, yt, src2d, tgt2d):
    D2 = ys.shape[1]
    rows_pad, width = src2d.shape
    e_pad = rows_pad * width

    mesh = plsc.VectorSubcoreMesh(core_axis_name="core",
                                  subcore_axis_name="subcore")

    NREP = 8
    NT = ys.shape[0] // NREP
    R = rows_pad // NTILES
    NBUF = 4

    cp = _sc_compiler_params()
    if "use_tc_tiling_on_sc" in pltpu.CompilerParams.__dataclass_fields__:
        cp = dataclasses.replace(cp, use_tc_tiling_on_sc=False)

    @functools.partial(
        pl.kernel, mesh=mesh, compiler_params=cp,
        out_type=(jax.ShapeDtypeStruct((e_pad, D2), jnp.int32),
                  jax.ShapeDtypeStruct((e_pad, D2), jnp.int32)),
        scratch_types=[
            pltpu.VMEM((R, width), jnp.int32),
            pltpu.VMEM((R, width), jnp.int32),
            pltpu.VMEM((NBUF, width, D2), jnp.int32),
            pltpu.SemaphoreType.DMA,
            pltpu.SemaphoreType.DMA,
        ],
    )
    def k(ys_hbm, yt_hbm, src_hbm, tgt_hbm, gs_hbm, gt_hbm,
          src_v, tgt_v, bufs, gsem, osem):
        cid = lax.axis_index("core")
        sid = lax.axis_index("subcore")
        w = sid * 2 + cid
        base = w * R

        pltpu.sync_copy(src_hbm.at[pl.ds(base, R)], src_v)
        pltpu.sync_copy(tgt_hbm.at[pl.ds(base, R)], tgt_v)

        off = jnp.int32(NT) * lax.rem(w, jnp.int32(NREP))

        @pl.loop(0, R)
        def _(r):
            @pl.loop(0, width, step=LANES)
            def _(c):
                src_v[r, pl.ds(c, LANES)] = src_v[r, pl.ds(c, LANES)] + off
                tgt_v[r, pl.ds(c, LANES)] = tgt_v[r, pl.ds(c, LANES)] + off

        def one_pass(tbl_hbm, idx_v, out_hbm):
            @pl.loop(0, R, step=NBUF)
            def _(g):
                hs = [pltpu.async_copy(tbl_hbm.at[idx_v.at[g + b]],
                                       bufs.at[b], gsem)
                      for b in range(NBUF)]
                for h in hs:
                    h.wait()
                row0 = (base + g) * width
                ss = [pltpu.async_copy(bufs.at[b],
                                       out_hbm.at[pl.ds(row0 + b * width,
                                                        width)], osem)
                      for b in range(NBUF)]
                for s in ss:
                    s.wait()

        one_pass(ys_hbm, src_v, gs_hbm)
        one_pass(yt_hbm, tgt_v, gt_hbm)

    return k(ys, yt, src2d, tgt2d)


def _tc_final_body(ea_ref, gs_ref, gt_ref, a_ref, we_ref, rw_ref, o_ref):
    D = ea_ref.shape[1]
    half = D // 2
    he = _hdot(ea_ref[...], we_ref[...], NT)
    s_lo, s_hi = _unpack_bf16_pair(gs_ref[...])
    t_lo, t_hi = _unpack_bf16_pair(gt_ref[...])
    a = a_ref[...]
    h_lo = (he[:, :half] + s_lo + t_lo) * a
    h_hi = (he[:, half:] + s_hi + t_hi) * a
    ssq = (jnp.sum(h_lo * h_lo, axis=1, keepdims=True)
           + jnp.sum(h_hi * h_hi, axis=1, keepdims=True))
    r = lax.rsqrt(ssq * jnp.float32(1.0 / D) + EPS)
    o_ref[:, :half] = h_lo * r * rw_ref[:, :half]
    o_ref[:, half:] = h_hi * r * rw_ref[:, half:]


def _tc_final(edge_attr, gs, gt, alpha, W_edge, rms_w2d):
    E, D = edge_attr.shape
    return pl.pallas_call(
        _tc_final_body,
        grid=(E // EBLK,),
        in_specs=[
            pl.BlockSpec((EBLK, D), lambda i: (i, 0)),
            pl.BlockSpec((EBLK, D // 2), lambda i: (i, 0)),
            pl.BlockSpec((EBLK, D // 2), lambda i: (i, 0)),
            pl.BlockSpec((EBLK, 1), lambda i: (i, 0)),
            pl.BlockSpec((D, D), lambda i: (0, 0)),
            pl.BlockSpec((1, D), lambda i: (0, 0)),
        ],
        out_specs=pl.BlockSpec((EBLK, D), lambda i: (i, 0)),
        out_shape=jax.ShapeDtypeStruct((E, D), jnp.float32),
    )(edge_attr, gs, gt, alpha, W_edge, rms_w2d)


def kernel(x_s, x_t, edge_index, edge_attr, x_u, W_src, W_tgt, W_edge,
           W_attn, rms_w):
    N, D = x_s.shape
    E = edge_attr.shape[0]
    src = edge_index[0].astype(jnp.int32)
    tgt = edge_index[1].astype(jnp.int32)

    npad = LANES * _round_up(_round_up(N, LANES) // LANES, LANES)
    rows = _round_up(E, D) // D
    rows_pad = NTILES * _round_up(_round_up(rows, NTILES) // NTILES, 8)
    e_pad = rows_pad * D

    ys, yt, es, et = _tc_prep(x_s, x_t, W_src, W_tgt, W_attn)
    ee = _tc_ee(edge_attr, W_edge, W_attn)

    es_t = jnp.pad(es[0], (0, npad - N))
    et_t = jnp.pad(et[0], (0, npad - N))
    ee2d = jnp.pad(ee[0], (0, e_pad - E)).reshape(rows_pad, D)
    src2d = jnp.pad(src, (0, e_pad - E)).reshape(rows_pad, D)
    tgt2d = jnp.pad(tgt, (0, e_pad - E)).reshape(rows_pad, D)

    p2d, dens = _sc_logits(es_t, et_t, ee2d, src2d, tgt2d, E)
    alpha2d = _sc_alpha(dens, p2d, src2d)
    ys_p = jnp.tile(jnp.pad(ys, ((0, npad - N), (0, 0))), (8, 1))
    yt_p = jnp.tile(jnp.pad(yt, ((0, npad - N), (0, 0))), (8, 1))
    gs, gt = _sc_gather(ys_p, yt_p, src2d, tgt2d)

    alpha = alpha2d.reshape(-1)[:E, None]
    return _tc_final(edge_attr, gs, gt, alpha, W_edge, rms_w.reshape(1, D))


# speedup vs baseline: 1.4350x; 1.4350x over previous
import dataclasses
import functools

import jax
import jax.numpy as jnp
import numpy as np
from jax import lax
from jax.experimental import pallas as pl
from jax.experimental.pallas import tpu as pltpu
from jax.experimental.pallas import tpu_sc as plsc

EPS = float(np.finfo(np.float32).eps)

NTILES = 32
LANES = 16
EBLK = 2560


def _round_up(x, m):
    return (x + m - 1) // m * m


def _sc_compiler_params():
    cp = pltpu.CompilerParams()
    if "needs_layout_passes" in pltpu.CompilerParams.__dataclass_fields__:
        cp = dataclasses.replace(cp, needs_layout_passes=False)
    return cp


NT = (((1,), (1,)), ((), ()))
NN = (((1,), (0,)), ((), ()))


def _hdot(a, b, dims):
    return lax.dot_general(a, b, dims, precision=lax.Precision.HIGHEST,
                           preferred_element_type=jnp.float32)


def _pack_bf16_pair(y):
    half = y.shape[1] // 2
    u = lax.bitcast_convert_type(y, jnp.uint32) + jnp.uint32(0x8000)
    lo = u[:, :half] >> jnp.uint32(16)
    hi = u[:, half:] & jnp.uint32(0xFFFF0000)
    return lax.bitcast_convert_type(lo | hi, jnp.int32)


def _unpack_bf16_pair(v):
    f_lo = lax.bitcast_convert_type(v << jnp.int32(16), jnp.float32)
    f_hi = lax.bitcast_convert_type(
        v & jnp.int32(np.int32(np.uint32(0xFFFF0000))), jnp.float32)
    return f_lo, f_hi


def _tc_prep_body(xs_ref, xt_ref, ws_ref, wt_ref, wa_ref,
                  ys_ref, yt_ref, es_ref, et_ref):
    ys = _hdot(xs_ref[...], ws_ref[...], NT)
    yt = _hdot(xt_ref[...], wt_ref[...], NT)
    ys_ref[...] = _pack_bf16_pair(ys)
    yt_ref[...] = _pack_bf16_pair(yt)
    es_ref[...] = _hdot(wa_ref[...], ys, NT)
    et_ref[...] = _hdot(wa_ref[...], yt, NT)


def _tc_prep(x_s, x_t, W_src, W_tgt, W_attn):
    N, D = x_s.shape
    full = lambda shape: pl.BlockSpec(shape, lambda: (0, 0))
    return pl.pallas_call(
        _tc_prep_body,
        in_specs=[full((N, D)), full((N, D)),
                  full((D, D)), full((D, D)), full((1, D))],
        out_specs=[full((N, D // 2)), full((N, D // 2)),
                   full((1, N)), full((1, N))],
        out_shape=[
            jax.ShapeDtypeStruct((N, D // 2), jnp.int32),
            jax.ShapeDtypeStruct((N, D // 2), jnp.int32),
            jax.ShapeDtypeStruct((1, N), jnp.float32),
            jax.ShapeDtypeStruct((1, N), jnp.float32),
        ],
    )(x_s, x_t, W_src, W_tgt, W_attn)


def _tc_ee_body(ea_ref, we_ref, wa_ref, ee_ref):
    c = _hdot(wa_ref[...], we_ref[...], NN)
    ee_ref[...] = _hdot(c, ea_ref[...], NT)


def _tc_ee(edge_attr, W_edge, W_attn):
    E, D = edge_attr.shape
    return pl.pallas_call(
        _tc_ee_body,
        grid=(E // EBLK,),
        in_specs=[
            pl.BlockSpec((EBLK, D), lambda i: (i, 0)),
            pl.BlockSpec((D, D), lambda i: (0, 0)),
            pl.BlockSpec((1, D), lambda i: (0, 0)),
        ],
        out_specs=pl.BlockSpec((1, EBLK), lambda i: (0, i)),
        out_shape=jax.ShapeDtypeStruct((1, E), jnp.float32),
    )(edge_attr, W_edge, W_attn)


def _sc_logits(es_t, et_t, ee2d, src2d, tgt2d, E):
    npad = es_t.shape[0]
    nslice = npad // LANES
    rows_pad, width = ee2d.shape
    R = rows_pad // NTILES

    mesh = plsc.VectorSubcoreMesh(core_axis_name="core",
                                  subcore_axis_name="subcore")

    @functools.partial(
        pl.kernel, mesh=mesh, compiler_params=_sc_compiler_params(),
        out_type=(jax.ShapeDtypeStruct((rows_pad, width), jnp.float32),
                  jax.ShapeDtypeStruct((2, npad), jnp.float32)),
        scratch_types=[
            pltpu.VMEM((npad,), jnp.float32),
            pltpu.VMEM((npad,), jnp.float32),
            pltpu.VMEM((R, width), jnp.float32),
            pltpu.VMEM((R, width), jnp.int32),
            pltpu.VMEM((R, width), jnp.int32),
            pltpu.VMEM((R, width), jnp.float32),
            pltpu.VMEM((nslice,), jnp.float32),
            pltpu.VMEM_SHARED((npad,), jnp.float32),
        ],
    )
    def k(es_hbm, et_hbm, ee_hbm, src_hbm, tgt_hbm, p_hbm, den_hbm,
          es_v, et_v, ee_v, src_v, tgt_v, p_v, zero_v, den_sh):
        cid = lax.axis_index("core")
        sid = lax.axis_index("subcore")
        w = sid * 2 + cid
        base = w * R

        pltpu.sync_copy(es_hbm, es_v)
        pltpu.sync_copy(et_hbm, et_v)
        pltpu.sync_copy(ee_hbm.at[pl.ds(base, R)], ee_v)
        pltpu.sync_copy(src_hbm.at[pl.ds(base, R)], src_v)
        pltpu.sync_copy(tgt_hbm.at[pl.ds(base, R)], tgt_v)

        @pl.loop(0, nslice, step=LANES)
        def _(i):
            zero_v[pl.ds(i, LANES)] = jnp.zeros((LANES,), jnp.float32)

        pltpu.sync_copy(zero_v, den_sh.at[pl.ds(sid * nslice, nslice)])
        plsc.subcore_barrier()

        @pl.loop(0, R)
        def _(r):
            @pl.loop(0, width, step=LANES)
            def _(col):
                si = src_v[r, pl.ds(col, LANES)]
                ti = tgt_v[r, pl.ds(col, LANES)]
                ev = ee_v[r, pl.ds(col, LANES)]
                e = plsc.load_gather(es_v, [si]) + plsc.load_gather(et_v, [ti]) + ev
                e = jnp.maximum(e, e * jnp.float32(0.2))
                pos = (base + r) * width + col + lax.iota(jnp.int32, LANES)
                p = jnp.where(pos < E, jnp.exp(e), jnp.float32(0.0))
                p_v[r, pl.ds(col, LANES)] = p

        @pl.loop(0, R)
        def _(r):
            pltpu.sync_copy(p_v.at[r], den_sh.at[src_v.at[r]], add=True)

        plsc.subcore_barrier()
        pltpu.sync_copy(p_v, p_hbm.at[pl.ds(base, R)])

        @pl.when(sid == 0)
        def _():
            pltpu.sync_copy(den_sh, den_hbm.at[cid])

    return k(es_t, et_t, ee2d, src2d, tgt2d)


def _sc_alpha(dens, p2d, src2d):
    npad = dens.shape[1]
    rows_pad, width = p2d.shape
    R = rows_pad // NTILES

    mesh = plsc.VectorSubcoreMesh(core_axis_name="core",
                                  subcore_axis_name="subcore")

    @functools.partial(
        pl.kernel, mesh=mesh, compiler_params=_sc_compiler_params(),
        out_type=jax.ShapeDtypeStruct((rows_pad, width), jnp.float32),
        scratch_types=[
            pltpu.VMEM((npad,), jnp.float32),
            pltpu.VMEM((npad,), jnp.float32),
            pltpu.VMEM((R, width), jnp.float32),
            pltpu.VMEM((R, width), jnp.int32),
            pltpu.VMEM((R, width), jnp.float32),
        ],
    )
    def k(den_hbm, p_hbm, src_hbm, a_hbm, d0_v, d1_v, p_v, src_v, a_v):
        cid = lax.axis_index("core")
        sid = lax.axis_index("subcore")
        w = sid * 2 + cid
        base = w * R

        pltpu.sync_copy(den_hbm.at[0], d0_v)
        pltpu.sync_copy(den_hbm.at[1], d1_v)
        pltpu.sync_copy(p_hbm.at[pl.ds(base, R)], p_v)
        pltpu.sync_copy(src_hbm.at[pl.ds(base, R)], src_v)

        @pl.loop(0, npad, step=LANES)
        def _(i):
            d0_v[pl.ds(i, LANES)] = d0_v[pl.ds(i, LANES)] + d1_v[pl.ds(i, LANES)]

        @pl.loop(0, R)
        def _(r):
            @pl.loop(0, width, step=LANES)
            def _(col):
                si = src_v[r, pl.ds(col, LANES)]
                d = plsc.load_gather(d0_v, [si])
                a_v[r, pl.ds(col, LANES)] = p_v[r, pl.ds(col, LANES)] / d

        pltpu.sync_copy(a_v, a_hbm.at[pl.ds(base, R)])

    return k(dens, p2d, src2d)


def _sc_gather(ys, yt, src2d, tgt2d):
    D2 = ys.shape[1]
    rows_pad, width = src2d.shape
    e_pad = rows_pad * width

    mesh = plsc.VectorSubcoreMesh(core_axis_name="core",
                                  subcore_axis_name="subcore")

    NT = ys.shape[0]
    SROWS = NT // 16
    R = rows_pad // NTILES
    NBUF = 4

    cp = _sc_compiler_params()
    if "use_tc_tiling_on_sc" in pltpu.CompilerParams.__dataclass_fields__:
        cp = dataclasses.replace(cp, use_tc_tiling_on_sc=False)

    @functools.partial(
        pl.kernel, mesh=mesh, compiler_params=cp,
        out_type=(jax.ShapeDtypeStruct((e_pad, D2), jnp.int32),
                  jax.ShapeDtypeStruct((e_pad, D2), jnp.int32)),
        scratch_types=[
            pltpu.VMEM((R, width), jnp.int32),
            pltpu.VMEM((R, width), jnp.int32),
            pltpu.VMEM((NBUF, width, D2), jnp.int32),
            pltpu.VMEM_SHARED((NT, D2), jnp.int32),
            pltpu.SemaphoreType.DMA,
            pltpu.SemaphoreType.DMA,
        ],
    )
    def k(ys_hbm, yt_hbm, src_hbm, tgt_hbm, gs_hbm, gt_hbm,
          src_v, tgt_v, bufs, tbl_sh, gsem, osem):
        cid = lax.axis_index("core")
        sid = lax.axis_index("subcore")
        w = sid * 2 + cid
        base = w * R

        pltpu.sync_copy(src_hbm.at[pl.ds(base, R)], src_v)
        pltpu.sync_copy(tgt_hbm.at[pl.ds(base, R)], tgt_v)

        def one_pass(tbl_hbm, idx_v, out_hbm):
            @pl.loop(0, SROWS, step=width)
            def _(c):
                row = sid * SROWS + c
                pltpu.sync_copy(tbl_hbm.at[pl.ds(row, width)], bufs.at[0])
                pltpu.sync_copy(bufs.at[0], tbl_sh.at[pl.ds(row, width)])

            plsc.subcore_barrier()

            @pl.loop(0, R, step=NBUF)
            def _(g):
                hs = [pltpu.async_copy(tbl_sh.at[idx_v.at[g + b]],
                                       bufs.at[b], gsem)
                      for b in range(NBUF)]
                for h in hs:
                    h.wait()
                row0 = (base + g) * width
                ss = [pltpu.async_copy(bufs.at[b],
                                       out_hbm.at[pl.ds(row0 + b * width,
                                                        width)], osem)
                      for b in range(NBUF)]
                for s in ss:
                    s.wait()

            plsc.subcore_barrier()

        one_pass(ys_hbm, src_v, gs_hbm)
        one_pass(yt_hbm, tgt_v, gt_hbm)

    return k(ys, yt, src2d, tgt2d)


def _tc_final_body(ea_ref, gs_ref, gt_ref, a_ref, we_ref, rw_ref, o_ref):
    D = ea_ref.shape[1]
    half = D // 2
    he = _hdot(ea_ref[...], we_ref[...], NT)
    s_lo, s_hi = _unpack_bf16_pair(gs_ref[...])
    t_lo, t_hi = _unpack_bf16_pair(gt_ref[...])
    a = a_ref[...]
    h_lo = (he[:, :half] + s_lo + t_lo) * a
    h_hi = (he[:, half:] + s_hi + t_hi) * a
    ssq = (jnp.sum(h_lo * h_lo, axis=1, keepdims=True)
           + jnp.sum(h_hi * h_hi, axis=1, keepdims=True))
    r = lax.rsqrt(ssq * jnp.float32(1.0 / D) + EPS)
    o_ref[:, :half] = h_lo * r * rw_ref[:, :half]
    o_ref[:, half:] = h_hi * r * rw_ref[:, half:]


def _tc_final(edge_attr, gs, gt, alpha, W_edge, rms_w2d):
    E, D = edge_attr.shape
    return pl.pallas_call(
        _tc_final_body,
        grid=(E // EBLK,),
        in_specs=[
            pl.BlockSpec((EBLK, D), lambda i: (i, 0)),
            pl.BlockSpec((EBLK, D // 2), lambda i: (i, 0)),
            pl.BlockSpec((EBLK, D // 2), lambda i: (i, 0)),
            pl.BlockSpec((EBLK, 1), lambda i: (i, 0)),
            pl.BlockSpec((D, D), lambda i: (0, 0)),
            pl.BlockSpec((1, D), lambda i: (0, 0)),
        ],
        out_specs=pl.BlockSpec((EBLK, D), lambda i: (i, 0)),
        out_shape=jax.ShapeDtypeStruct((E, D), jnp.float32),
    )(edge_attr, gs, gt, alpha, W_edge, rms_w2d)


def kernel(x_s, x_t, edge_index, edge_attr, x_u, W_src, W_tgt, W_edge,
           W_attn, rms_w):
    N, D = x_s.shape
    E = edge_attr.shape[0]
    src = edge_index[0].astype(jnp.int32)
    tgt = edge_index[1].astype(jnp.int32)

    npad = LANES * _round_up(_round_up(N, LANES) // LANES, LANES)
    rows = _round_up(E, D) // D
    rows_pad = NTILES * _round_up(_round_up(rows, NTILES) // NTILES, 8)
    e_pad = rows_pad * D

    ys, yt, es, et = _tc_prep(x_s, x_t, W_src, W_tgt, W_attn)
    ee = _tc_ee(edge_attr, W_edge, W_attn)

    es_t = jnp.pad(es[0], (0, npad - N))
    et_t = jnp.pad(et[0], (0, npad - N))
    ee2d = jnp.pad(ee[0], (0, e_pad - E)).reshape(rows_pad, D)
    src2d = jnp.pad(src, (0, e_pad - E)).reshape(rows_pad, D)
    tgt2d = jnp.pad(tgt, (0, e_pad - E)).reshape(rows_pad, D)

    p2d, dens = _sc_logits(es_t, et_t, ee2d, src2d, tgt2d, E)
    alpha2d = _sc_alpha(dens, p2d, src2d)
    ys_p = jnp.pad(ys, ((0, npad - N), (0, 0)))
    yt_p = jnp.pad(yt, ((0, npad - N), (0, 0)))
    gs, gt = _sc_gather(ys_p, yt_p, src2d, tgt2d)

    alpha = alpha2d.reshape(-1)[:E, None]
    return _tc_final(edge_attr, gs, gt, alpha, W_edge, rms_w.reshape(1, D))


# speedup vs baseline: 2.0440x; 1.4244x over previous
import dataclasses
import functools

import jax
import jax.numpy as jnp
import numpy as np
from jax import lax
from jax.experimental import pallas as pl
from jax.experimental.pallas import tpu as pltpu
from jax.experimental.pallas import tpu_sc as plsc

EPS = float(np.finfo(np.float32).eps)

NTILES = 32
LANES = 16
EBLK = 2560


def _round_up(x, m):
    return (x + m - 1) // m * m


def _sc_compiler_params():
    cp = pltpu.CompilerParams()
    if "needs_layout_passes" in pltpu.CompilerParams.__dataclass_fields__:
        cp = dataclasses.replace(cp, needs_layout_passes=False)
    return cp


NT = (((1,), (1,)), ((), ()))
NN = (((1,), (0,)), ((), ()))


def _hdot(a, b, dims):
    return lax.dot_general(a, b, dims, precision=lax.Precision.HIGHEST,
                           preferred_element_type=jnp.float32)


def _pack_bf16_pair(y):
    half = y.shape[1] // 2
    u = lax.bitcast_convert_type(y, jnp.uint32) + jnp.uint32(0x8000)
    lo = u[:, :half] >> jnp.uint32(16)
    hi = u[:, half:] & jnp.uint32(0xFFFF0000)
    return lax.bitcast_convert_type(lo | hi, jnp.int32)


def _unpack_bf16_pair(v):
    f_lo = lax.bitcast_convert_type(v << jnp.int32(16), jnp.float32)
    f_hi = lax.bitcast_convert_type(
        v & jnp.int32(np.int32(np.uint32(0xFFFF0000))), jnp.float32)
    return f_lo, f_hi


def _tc_prep_body(xs_ref, xt_ref, ws_ref, wt_ref, wa_ref,
                  ys_ref, yt_ref, es_ref, et_ref):
    ys = _hdot(xs_ref[...], ws_ref[...], NT)
    yt = _hdot(xt_ref[...], wt_ref[...], NT)
    ys_ref[...] = ys
    yt_ref[...] = yt
    es_ref[...] = _hdot(wa_ref[...], ys, NT)
    et_ref[...] = _hdot(wa_ref[...], yt, NT)


def _tc_prep(x_s, x_t, W_src, W_tgt, W_attn):
    N, D = x_s.shape
    full = lambda shape: pl.BlockSpec(shape, lambda: (0, 0))
    return pl.pallas_call(
        _tc_prep_body,
        in_specs=[full((N, D)), full((N, D)),
                  full((D, D)), full((D, D)), full((1, D))],
        out_specs=[full((N, D)), full((N, D)), full((1, N)), full((1, N))],
        out_shape=[
            jax.ShapeDtypeStruct((N, D), jnp.float32),
            jax.ShapeDtypeStruct((N, D), jnp.float32),
            jax.ShapeDtypeStruct((1, N), jnp.float32),
            jax.ShapeDtypeStruct((1, N), jnp.float32),
        ],
    )(x_s, x_t, W_src, W_tgt, W_attn)


def _tc_ee_body(ea_ref, we_ref, wa_ref, ee_ref):
    c = _hdot(wa_ref[...], we_ref[...], NN)
    ee_ref[...] = _hdot(c, ea_ref[...], NT)


def _tc_ee(edge_attr, W_edge, W_attn):
    E, D = edge_attr.shape
    return pl.pallas_call(
        _tc_ee_body,
        grid=(E // EBLK,),
        in_specs=[
            pl.BlockSpec((EBLK, D), lambda i: (i, 0)),
            pl.BlockSpec((D, D), lambda i: (0, 0)),
            pl.BlockSpec((1, D), lambda i: (0, 0)),
        ],
        out_specs=pl.BlockSpec((1, EBLK), lambda i: (0, i)),
        out_shape=jax.ShapeDtypeStruct((1, E), jnp.float32),
    )(edge_attr, W_edge, W_attn)


def _sc_logits(es_t, et_t, ee2d, src2d, tgt2d, E):
    npad = es_t.shape[0]
    nslice = npad // LANES
    rows_pad, width = ee2d.shape
    R = rows_pad // NTILES

    mesh = plsc.VectorSubcoreMesh(core_axis_name="core",
                                  subcore_axis_name="subcore")

    @functools.partial(
        pl.kernel, mesh=mesh, compiler_params=_sc_compiler_params(),
        out_type=(jax.ShapeDtypeStruct((rows_pad, width), jnp.float32),
                  jax.ShapeDtypeStruct((2, npad), jnp.float32)),
        scratch_types=[
            pltpu.VMEM((npad,), jnp.float32),
            pltpu.VMEM((npad,), jnp.float32),
            pltpu.VMEM((R, width), jnp.float32),
            pltpu.VMEM((R, width), jnp.int32),
            pltpu.VMEM((R, width), jnp.int32),
            pltpu.VMEM((R, width), jnp.float32),
            pltpu.VMEM((nslice,), jnp.float32),
            pltpu.VMEM_SHARED((npad,), jnp.float32),
        ],
    )
    def k(es_hbm, et_hbm, ee_hbm, src_hbm, tgt_hbm, p_hbm, den_hbm,
          es_v, et_v, ee_v, src_v, tgt_v, p_v, zero_v, den_sh):
        cid = lax.axis_index("core")
        sid = lax.axis_index("subcore")
        w = sid * 2 + cid
        base = w * R

        pltpu.sync_copy(es_hbm, es_v)
        pltpu.sync_copy(et_hbm, et_v)
        pltpu.sync_copy(ee_hbm.at[pl.ds(base, R)], ee_v)
        pltpu.sync_copy(src_hbm.at[pl.ds(base, R)], src_v)
        pltpu.sync_copy(tgt_hbm.at[pl.ds(base, R)], tgt_v)

        @pl.loop(0, nslice, step=LANES)
        def _(i):
            zero_v[pl.ds(i, LANES)] = jnp.zeros((LANES,), jnp.float32)

        pltpu.sync_copy(zero_v, den_sh.at[pl.ds(sid * nslice, nslice)])
        plsc.subcore_barrier()

        @pl.loop(0, R)
        def _(r):
            @pl.loop(0, width, step=LANES)
            def _(col):
                si = src_v[r, pl.ds(col, LANES)]
                ti = tgt_v[r, pl.ds(col, LANES)]
                ev = ee_v[r, pl.ds(col, LANES)]
                e = plsc.load_gather(es_v, [si]) + plsc.load_gather(et_v, [ti]) + ev
                e = jnp.maximum(e, e * jnp.float32(0.2))
                pos = (base + r) * width + col + lax.iota(jnp.int32, LANES)
                p = jnp.where(pos < E, jnp.exp(e), jnp.float32(0.0))
                p_v[r, pl.ds(col, LANES)] = p

        @pl.loop(0, R)
        def _(r):
            pltpu.sync_copy(p_v.at[r], den_sh.at[src_v.at[r]], add=True)

        plsc.subcore_barrier()
        pltpu.sync_copy(p_v, p_hbm.at[pl.ds(base, R)])

        @pl.when(sid == 0)
        def _():
            pltpu.sync_copy(den_sh, den_hbm.at[cid])

    return k(es_t, et_t, ee2d, src2d, tgt2d)


def _sc_alpha(dens, p2d, src2d):
    npad = dens.shape[1]
    rows_pad, width = p2d.shape
    R = rows_pad // NTILES

    mesh = plsc.VectorSubcoreMesh(core_axis_name="core",
                                  subcore_axis_name="subcore")

    @functools.partial(
        pl.kernel, mesh=mesh, compiler_params=_sc_compiler_params(),
        out_type=jax.ShapeDtypeStruct((rows_pad, width), jnp.float32),
        scratch_types=[
            pltpu.VMEM((npad,), jnp.float32),
            pltpu.VMEM((npad,), jnp.float32),
            pltpu.VMEM((R, width), jnp.float32),
            pltpu.VMEM((R, width), jnp.int32),
            pltpu.VMEM((R, width), jnp.float32),
        ],
    )
    def k(den_hbm, p_hbm, src_hbm, a_hbm, d0_v, d1_v, p_v, src_v, a_v):
        cid = lax.axis_index("core")
        sid = lax.axis_index("subcore")
        w = sid * 2 + cid
        base = w * R

        pltpu.sync_copy(den_hbm.at[0], d0_v)
        pltpu.sync_copy(den_hbm.at[1], d1_v)
        pltpu.sync_copy(p_hbm.at[pl.ds(base, R)], p_v)
        pltpu.sync_copy(src_hbm.at[pl.ds(base, R)], src_v)

        @pl.loop(0, npad, step=LANES)
        def _(i):
            d0_v[pl.ds(i, LANES)] = d0_v[pl.ds(i, LANES)] + d1_v[pl.ds(i, LANES)]

        @pl.loop(0, R)
        def _(r):
            @pl.loop(0, width, step=LANES)
            def _(col):
                si = src_v[r, pl.ds(col, LANES)]
                d = plsc.load_gather(d0_v, [si])
                a_v[r, pl.ds(col, LANES)] = p_v[r, pl.ds(col, LANES)] / d

        pltpu.sync_copy(a_v, a_hbm.at[pl.ds(base, R)])

    return k(dens, p2d, src2d)


def _sc_gather(ys, yt, src2d, tgt2d):
    D2 = ys.shape[1]
    rows_pad, width = src2d.shape
    e_pad = rows_pad * width

    mesh = plsc.VectorSubcoreMesh(core_axis_name="core",
                                  subcore_axis_name="subcore")

    NT = ys.shape[0]
    SROWS = NT // 16
    R = rows_pad // NTILES
    NBUF = 2

    @functools.partial(
        pl.kernel, mesh=mesh, compiler_params=_sc_compiler_params(),
        out_type=(jax.ShapeDtypeStruct((e_pad, D2), jnp.float32),
                  jax.ShapeDtypeStruct((e_pad, D2), jnp.float32)),
        scratch_types=[
            pltpu.VMEM((R, width), jnp.int32),
            pltpu.VMEM((NBUF, width, D2), jnp.float32),
            pltpu.VMEM_SHARED((NT, D2), jnp.float32),
            pltpu.SemaphoreType.DMA,
            pltpu.SemaphoreType.DMA,
        ],
    )
    def k(ys_hbm, yt_hbm, src_hbm, tgt_hbm, gs_hbm, gt_hbm,
          idx_v, bufs, tbl_sh, gsem, osem):
        cid = lax.axis_index("core")
        sid = lax.axis_index("subcore")
        w = sid * 2 + cid
        base = w * R

        def one_pass(tbl_hbm, eidx_hbm, out_hbm):
            pltpu.sync_copy(eidx_hbm.at[pl.ds(base, R)], idx_v)
            @pl.loop(0, SROWS, step=width)
            def _(c):
                row = sid * SROWS + c
                pltpu.sync_copy(tbl_hbm.at[pl.ds(row, width)], bufs.at[0])
                pltpu.sync_copy(bufs.at[0], tbl_sh.at[pl.ds(row, width)])

            plsc.subcore_barrier()

            @pl.loop(0, R, step=NBUF)
            def _(g):
                hs = [pltpu.async_copy(tbl_sh.at[idx_v.at[g + b]],
                                       bufs.at[b], gsem)
                      for b in range(NBUF)]
                for h in hs:
                    h.wait()
                row0 = (base + g) * width
                ss = [pltpu.async_copy(bufs.at[b],
                                       out_hbm.at[pl.ds(row0 + b * width,
                                                        width)], osem)
                      for b in range(NBUF)]
                for s in ss:
                    s.wait()

            plsc.subcore_barrier()

        one_pass(ys_hbm, src_hbm, gs_hbm)
        one_pass(yt_hbm, tgt_hbm, gt_hbm)

    return k(ys, yt, src2d, tgt2d)


def _tc_final_body(ea_ref, gs_ref, gt_ref, a_ref, we_ref, rw_ref, o_ref):
    he = _hdot(ea_ref[...], we_ref[...], NT)
    h = (he + gs_ref[...] + gt_ref[...]) * a_ref[...]
    ms = jnp.mean(h * h, axis=1, keepdims=True)
    o_ref[...] = h * lax.rsqrt(ms + EPS) * rw_ref[...]


def _tc_final(edge_attr, gs, gt, alpha, W_edge, rms_w2d):
    E, D = edge_attr.shape
    return pl.pallas_call(
        _tc_final_body,
        grid=(E // EBLK,),
        in_specs=[
            pl.BlockSpec((EBLK, D), lambda i: (i, 0)),
            pl.BlockSpec((EBLK, D), lambda i: (i, 0)),
            pl.BlockSpec((EBLK, D), lambda i: (i, 0)),
            pl.BlockSpec((EBLK, 1), lambda i: (i, 0)),
            pl.BlockSpec((D, D), lambda i: (0, 0)),
            pl.BlockSpec((1, D), lambda i: (0, 0)),
        ],
        out_specs=pl.BlockSpec((EBLK, D), lambda i: (i, 0)),
        out_shape=jax.ShapeDtypeStruct((E, D), jnp.float32),
    )(edge_attr, gs, gt, alpha, W_edge, rms_w2d)


def kernel(x_s, x_t, edge_index, edge_attr, x_u, W_src, W_tgt, W_edge,
           W_attn, rms_w):
    N, D = x_s.shape
    E = edge_attr.shape[0]
    src = edge_index[0].astype(jnp.int32)
    tgt = edge_index[1].astype(jnp.int32)

    npad = LANES * _round_up(_round_up(N, LANES) // LANES, LANES)
    rows = _round_up(E, D) // D
    rows_pad = NTILES * _round_up(_round_up(rows, NTILES) // NTILES, 8)
    e_pad = rows_pad * D

    ys, yt, es, et = _tc_prep(x_s, x_t, W_src, W_tgt, W_attn)
    ee = _tc_ee(edge_attr, W_edge, W_attn)

    es_t = jnp.pad(es[0], (0, npad - N))
    et_t = jnp.pad(et[0], (0, npad - N))
    ee2d = jnp.pad(ee[0], (0, e_pad - E)).reshape(rows_pad, D)
    src2d = jnp.pad(src, (0, e_pad - E)).reshape(rows_pad, D)
    tgt2d = jnp.pad(tgt, (0, e_pad - E)).reshape(rows_pad, D)

    p2d, dens = _sc_logits(es_t, et_t, ee2d, src2d, tgt2d, E)
    alpha2d = _sc_alpha(dens, p2d, src2d)
    ys_p = jnp.pad(ys, ((0, npad - N), (0, 0)))
    yt_p = jnp.pad(yt, ((0, npad - N), (0, 0)))
    gs, gt = _sc_gather(ys_p, yt_p, src2d, tgt2d)

    alpha = alpha2d.reshape(-1)[:E, None]
    return _tc_final(edge_attr, gs, gt, alpha, W_edge, rms_w.reshape(1, D))


# speedup vs baseline: 2.1993x; 1.0759x over previous
import dataclasses
import functools

import jax
import jax.numpy as jnp
import numpy as np
from jax import lax
from jax.experimental import pallas as pl
from jax.experimental.pallas import tpu as pltpu
from jax.experimental.pallas import tpu_sc as plsc

EPS = float(np.finfo(np.float32).eps)

NTILES = 32
LANES = 16
EBLK = 2560


def _round_up(x, m):
    return (x + m - 1) // m * m


def _sc_compiler_params():
    cp = pltpu.CompilerParams()
    if "needs_layout_passes" in pltpu.CompilerParams.__dataclass_fields__:
        cp = dataclasses.replace(cp, needs_layout_passes=False)
    return cp


NT = (((1,), (1,)), ((), ()))
NN = (((1,), (0,)), ((), ()))


def _hdot(a, b, dims):
    return lax.dot_general(a, b, dims, precision=lax.Precision.HIGHEST,
                           preferred_element_type=jnp.float32)


def _pack_bf16_pair(y):
    half = y.shape[1] // 2
    u = lax.bitcast_convert_type(y, jnp.uint32) + jnp.uint32(0x8000)
    lo = u[:, :half] >> jnp.uint32(16)
    hi = u[:, half:] & jnp.uint32(0xFFFF0000)
    return lax.bitcast_convert_type(lo | hi, jnp.int32)


def _unpack_bf16_pair(v):
    f_lo = lax.bitcast_convert_type(v << jnp.int32(16), jnp.float32)
    f_hi = lax.bitcast_convert_type(
        v & jnp.int32(np.int32(np.uint32(0xFFFF0000))), jnp.float32)
    return f_lo, f_hi


def _tc_prep_body(xs_ref, xt_ref, ws_ref, wt_ref, we_ref, wa_ref, ea_ref,
                  ys_ref, yt_ref, es_ref, et_ref, ee_ref):
    @pl.when(pl.program_id(0) == 0)
    def _():
        ys = _hdot(xs_ref[...], ws_ref[...], NT)
        yt = _hdot(xt_ref[...], wt_ref[...], NT)
        ys_ref[...] = ys
        yt_ref[...] = yt
        es_ref[...] = _hdot(wa_ref[...], ys, NT)
        et_ref[...] = _hdot(wa_ref[...], yt, NT)

    c = lax.dot_general(wa_ref[...], we_ref[...], NN,
                        preferred_element_type=jnp.float32)
    ee_ref[...] = lax.dot_general(c, ea_ref[...], NT,
                                  preferred_element_type=jnp.float32)


def _tc_prep(x_s, x_t, W_src, W_tgt, W_edge, W_attn, edge_attr):
    N, D = x_s.shape
    E = edge_attr.shape[0]
    full = lambda shape: pl.BlockSpec(shape, lambda i: (0, 0))
    return pl.pallas_call(
        _tc_prep_body,
        grid=(E // EBLK,),
        in_specs=[full((N, D)), full((N, D)),
                  full((D, D)), full((D, D)), full((D, D)), full((1, D)),
                  pl.BlockSpec((EBLK, D), lambda i: (i, 0))],
        out_specs=[full((N, D)), full((N, D)), full((1, N)), full((1, N)),
                   pl.BlockSpec((1, EBLK), lambda i: (0, i))],
        out_shape=[
            jax.ShapeDtypeStruct((N, D), jnp.float32),
            jax.ShapeDtypeStruct((N, D), jnp.float32),
            jax.ShapeDtypeStruct((1, N), jnp.float32),
            jax.ShapeDtypeStruct((1, N), jnp.float32),
            jax.ShapeDtypeStruct((1, E), jnp.float32),
        ],
    )(x_s, x_t, W_src, W_tgt, W_edge, W_attn, edge_attr)


def _sc_logits(es_t, et_t, ee2d, src2d, tgt2d, E):
    npad = es_t.shape[0]
    nslice = npad // LANES
    rows_pad, width = ee2d.shape
    R = rows_pad // NTILES

    mesh = plsc.VectorSubcoreMesh(core_axis_name="core",
                                  subcore_axis_name="subcore")

    @functools.partial(
        pl.kernel, mesh=mesh, compiler_params=_sc_compiler_params(),
        out_type=(jax.ShapeDtypeStruct((rows_pad, width), jnp.float32),
                  jax.ShapeDtypeStruct((2, npad), jnp.float32)),
        scratch_types=[
            pltpu.VMEM((npad,), jnp.float32),
            pltpu.VMEM((npad,), jnp.float32),
            pltpu.VMEM((R, width), jnp.float32),
            pltpu.VMEM((R, width), jnp.int32),
            pltpu.VMEM((R, width), jnp.int32),
            pltpu.VMEM((R, width), jnp.float32),
            pltpu.VMEM((nslice,), jnp.float32),
            pltpu.VMEM_SHARED((npad,), jnp.float32),
        ],
    )
    def k(es_hbm, et_hbm, ee_hbm, src_hbm, tgt_hbm, p_hbm, den_hbm,
          es_v, et_v, ee_v, src_v, tgt_v, p_v, zero_v, den_sh):
        cid = lax.axis_index("core")
        sid = lax.axis_index("subcore")
        w = sid * 2 + cid
        base = w * R

        pltpu.sync_copy(es_hbm, es_v)
        pltpu.sync_copy(et_hbm, et_v)
        pltpu.sync_copy(ee_hbm.at[pl.ds(base, R)], ee_v)
        pltpu.sync_copy(src_hbm.at[pl.ds(base, R)], src_v)
        pltpu.sync_copy(tgt_hbm.at[pl.ds(base, R)], tgt_v)

        @pl.loop(0, nslice, step=LANES)
        def _(i):
            zero_v[pl.ds(i, LANES)] = jnp.zeros((LANES,), jnp.float32)

        pltpu.sync_copy(zero_v, den_sh.at[pl.ds(sid * nslice, nslice)])
        plsc.subcore_barrier()

        @pl.loop(0, R)
        def _(r):
            @pl.loop(0, width, step=LANES)
            def _(col):
                si = src_v[r, pl.ds(col, LANES)]
                ti = tgt_v[r, pl.ds(col, LANES)]
                ev = ee_v[r, pl.ds(col, LANES)]
                e = plsc.load_gather(es_v, [si]) + plsc.load_gather(et_v, [ti]) + ev
                e = jnp.maximum(e, e * jnp.float32(0.2))
                pos = (base + r) * width + col + lax.iota(jnp.int32, LANES)
                p = jnp.where(pos < E, jnp.exp(e), jnp.float32(0.0))
                p_v[r, pl.ds(col, LANES)] = p

        @pl.loop(0, R)
        def _(r):
            pltpu.sync_copy(p_v.at[r], den_sh.at[src_v.at[r]], add=True)

        plsc.subcore_barrier()
        pltpu.sync_copy(p_v, p_hbm.at[pl.ds(base, R)])

        @pl.when(sid == 0)
        def _():
            pltpu.sync_copy(den_sh, den_hbm.at[cid])

    return k(es_t, et_t, ee2d, src2d, tgt2d)


def _sc_alpha(dens, p2d, src2d):
    npad = dens.shape[1]
    rows_pad, width = p2d.shape
    R = rows_pad // NTILES

    mesh = plsc.VectorSubcoreMesh(core_axis_name="core",
                                  subcore_axis_name="subcore")

    @functools.partial(
        pl.kernel, mesh=mesh, compiler_params=_sc_compiler_params(),
        out_type=jax.ShapeDtypeStruct((rows_pad, width), jnp.float32),
        scratch_types=[
            pltpu.VMEM((npad,), jnp.float32),
            pltpu.VMEM((npad,), jnp.float32),
            pltpu.VMEM((R, width), jnp.float32),
            pltpu.VMEM((R, width), jnp.int32),
            pltpu.VMEM((R, width), jnp.float32),
        ],
    )
    def k(den_hbm, p_hbm, src_hbm, a_hbm, d0_v, d1_v, p_v, src_v, a_v):
        cid = lax.axis_index("core")
        sid = lax.axis_index("subcore")
        w = sid * 2 + cid
        base = w * R

        pltpu.sync_copy(den_hbm.at[0], d0_v)
        pltpu.sync_copy(den_hbm.at[1], d1_v)
        pltpu.sync_copy(p_hbm.at[pl.ds(base, R)], p_v)
        pltpu.sync_copy(src_hbm.at[pl.ds(base, R)], src_v)

        @pl.loop(0, npad, step=LANES)
        def _(i):
            d0_v[pl.ds(i, LANES)] = d0_v[pl.ds(i, LANES)] + d1_v[pl.ds(i, LANES)]

        @pl.loop(0, R)
        def _(r):
            @pl.loop(0, width, step=LANES)
            def _(col):
                si = src_v[r, pl.ds(col, LANES)]
                d = plsc.load_gather(d0_v, [si])
                a_v[r, pl.ds(col, LANES)] = p_v[r, pl.ds(col, LANES)] / d

        pltpu.sync_copy(a_v, a_hbm.at[pl.ds(base, R)])

    return k(dens, p2d, src2d)


def _sc_gather(ys, yt, src2d, tgt2d):
    D2 = ys.shape[1]
    rows_pad, width = src2d.shape
    e_pad = rows_pad * width

    mesh = plsc.VectorSubcoreMesh(core_axis_name="core",
                                  subcore_axis_name="subcore")

    NT = ys.shape[0]
    SROWS = NT // 16
    R = rows_pad // NTILES
    NBUF = 2

    @functools.partial(
        pl.kernel, mesh=mesh, compiler_params=_sc_compiler_params(),
        out_type=(jax.ShapeDtypeStruct((e_pad, D2), jnp.float32),
                  jax.ShapeDtypeStruct((e_pad, D2), jnp.float32)),
        scratch_types=[
            pltpu.VMEM((R, width), jnp.int32),
            pltpu.VMEM((NBUF, width, D2), jnp.float32),
            pltpu.VMEM_SHARED((NT, D2), jnp.float32),
            pltpu.SemaphoreType.DMA,
            pltpu.SemaphoreType.DMA,
        ],
    )
    def k(ys_hbm, yt_hbm, src_hbm, tgt_hbm, gs_hbm, gt_hbm,
          idx_v, bufs, tbl_sh, gsem, osem):
        cid = lax.axis_index("core")
        sid = lax.axis_index("subcore")
        w = sid * 2 + cid
        base = w * R

        def one_pass(tbl_hbm, eidx_hbm, out_hbm):
            pltpu.sync_copy(eidx_hbm.at[pl.ds(base, R)], idx_v)
            @pl.loop(0, SROWS, step=width)
            def _(c):
                row = sid * SROWS + c
                pltpu.sync_copy(tbl_hbm.at[pl.ds(row, width)], bufs.at[0])
                pltpu.sync_copy(bufs.at[0], tbl_sh.at[pl.ds(row, width)])

            plsc.subcore_barrier()

            @pl.loop(0, R, step=NBUF)
            def _(g):
                hs = [pltpu.async_copy(tbl_sh.at[idx_v.at[g + b]],
                                       bufs.at[b], gsem)
                      for b in range(NBUF)]
                for h in hs:
                    h.wait()
                row0 = (base + g) * width
                ss = [pltpu.async_copy(bufs.at[b],
                                       out_hbm.at[pl.ds(row0 + b * width,
                                                        width)], osem)
                      for b in range(NBUF)]
                for s in ss:
                    s.wait()

            plsc.subcore_barrier()

        one_pass(ys_hbm, src_hbm, gs_hbm)
        one_pass(yt_hbm, tgt_hbm, gt_hbm)

    return k(ys, yt, src2d, tgt2d)


def _tc_final_body(ea_ref, gs_ref, gt_ref, a_ref, we_ref, rw_ref, o_ref):
    he = _hdot(ea_ref[...], we_ref[...], NT)
    h = (he + gs_ref[...] + gt_ref[...]) * a_ref[...]
    ms = jnp.mean(h * h, axis=1, keepdims=True)
    o_ref[...] = h * lax.rsqrt(ms + EPS) * rw_ref[...]


def _tc_final(edge_attr, gs, gt, alpha, W_edge, rms_w2d):
    E, D = edge_attr.shape
    return pl.pallas_call(
        _tc_final_body,
        grid=(E // EBLK,),
        in_specs=[
            pl.BlockSpec((EBLK, D), lambda i: (i, 0)),
            pl.BlockSpec((EBLK, D), lambda i: (i, 0)),
            pl.BlockSpec((EBLK, D), lambda i: (i, 0)),
            pl.BlockSpec((EBLK, 1), lambda i: (i, 0)),
            pl.BlockSpec((D, D), lambda i: (0, 0)),
            pl.BlockSpec((1, D), lambda i: (0, 0)),
        ],
        out_specs=pl.BlockSpec((EBLK, D), lambda i: (i, 0)),
        out_shape=jax.ShapeDtypeStruct((E, D), jnp.float32),
    )(edge_attr, gs, gt, alpha, W_edge, rms_w2d)


def kernel(x_s, x_t, edge_index, edge_attr, x_u, W_src, W_tgt, W_edge,
           W_attn, rms_w):
    N, D = x_s.shape
    E = edge_attr.shape[0]
    src = edge_index[0].astype(jnp.int32)
    tgt = edge_index[1].astype(jnp.int32)

    npad = LANES * _round_up(_round_up(N, LANES) // LANES, LANES)
    rows = _round_up(E, D) // D
    rows_pad = NTILES * _round_up(_round_up(rows, NTILES) // NTILES, 8)
    e_pad = rows_pad * D

    ys, yt, es, et, ee = _tc_prep(x_s, x_t, W_src, W_tgt, W_edge, W_attn,
                                  edge_attr)

    es_t = jnp.pad(es[0], (0, npad - N))
    et_t = jnp.pad(et[0], (0, npad - N))
    ee2d = jnp.pad(ee[0], (0, e_pad - E)).reshape(rows_pad, D)
    src2d = jnp.pad(src, (0, e_pad - E)).reshape(rows_pad, D)
    tgt2d = jnp.pad(tgt, (0, e_pad - E)).reshape(rows_pad, D)

    p2d, dens = _sc_logits(es_t, et_t, ee2d, src2d, tgt2d, E)
    alpha2d = _sc_alpha(dens, p2d, src2d)
    ys_p = jnp.pad(ys, ((0, npad - N), (0, 0)))
    yt_p = jnp.pad(yt, ((0, npad - N), (0, 0)))
    gs, gt = _sc_gather(ys_p, yt_p, src2d, tgt2d)

    alpha = alpha2d.reshape(-1)[:E, None]
    return _tc_final(edge_attr, gs, gt, alpha, W_edge, rms_w.reshape(1, D))
